# Initial kernel scaffold; baseline (speedup 1.0000x reference)
import functools

import jax
import jax.numpy as jnp
from jax import lax
from jax.experimental import pallas as pl
from jax.experimental.pallas import tpu as pltpu
from jax.experimental.pallas import tpu_sc as plsc

N = 10000
D = 128
E = 320000
K_HOPS = 3

NC = 2
NS = 16
NW = NC * NS

EPW = E // NW
CHUNK = 125
NCHUNK = EPW // CHUNK

NP = 10240
DEGW = NP // NW

ROWS_PER_TILE = N // NS
ZROWS = 125


def _mesh():
  return plsc.VectorSubcoreMesh(
      core_axis_name="c", subcore_axis_name="s", num_cores=NC, num_subcores=NS
  )


def _zero_fill(vref, nwords):
  z = jnp.zeros((16,), jnp.float32)

  def body(i, carry):
    vref[pl.ds(i * 16, 16)] = z
    return carry

  lax.fori_loop(0, nwords // 16, body, 0)


def _deg_body(dst_hbm, degp_hbm, idx_v, ones_v, deg_sh):
  sid = lax.axis_index("s")
  cid = lax.axis_index("c")
  wid = cid * NS + sid

  zslice = NP // NS
  _zero_fill(ones_v, zslice)
  pltpu.sync_copy(ones_v.at[pl.ds(0, zslice)],
                  deg_sh.at[pl.ds(sid * zslice, zslice)])

  one = jnp.ones((16,), jnp.float32)

  def fill_ones(i, carry):
    ones_v[pl.ds(i * 16, 16)] = one
    return carry

  lax.fori_loop(0, EPW // 16, fill_ones, 0)

  pltpu.sync_copy(dst_hbm.at[pl.ds(wid * EPW, EPW)], idx_v)
  plsc.subcore_barrier()

  pltpu.sync_copy(ones_v, deg_sh.at[idx_v], add=True)
  plsc.subcore_barrier()

  pltpu.sync_copy(
      deg_sh.at[pl.ds(sid * zslice, zslice)],
      degp_hbm.at[cid, pl.ds(sid * zslice, zslice)],
  )


def _k_deg(dst):
  f = pl.kernel(
      _deg_body,
      out_type=jax.ShapeDtypeStruct((NC, NP), jnp.float32),
      mesh=_mesh(),
      scratch_types=[
          pltpu.VMEM((EPW,), jnp.int32),
          pltpu.VMEM((EPW,), jnp.float32),
          pltpu.VMEM_SHARED((NP,), jnp.float32),
      ],
  )
  return f(dst)


def _rsqrt16(x):
  i = plsc.bitcast(x, jnp.int32)
  i = jnp.int32(0x5F3759DF) - (i >> 1)
  r = plsc.bitcast(i, jnp.float32)
  for _ in range(3):
    r = r * (1.5 - 0.5 * x * r * r)
  return r


def _prep_body(degp_hbm, y_hbm, d_hbm, d2_hbm, g0_hbm,
               p0_v, p1_v, d_v, d2_v, row_v, sem):
  sid = lax.axis_index("s")
  cid = lax.axis_index("c")
  wid = cid * NS + sid
  base = wid * DEGW

  pltpu.sync_copy(degp_hbm.at[0, pl.ds(base, DEGW)], p0_v)
  pltpu.sync_copy(degp_hbm.at[1, pl.ds(base, DEGW)], p1_v)

  def chunk(i, carry):
    deg = p0_v[pl.ds(i * 16, 16)] + p1_v[pl.ds(i * 16, 16)] + 1.0
    d_v[pl.ds(i * 16, 16)] = _rsqrt16(deg)
    d2_v[pl.ds(i * 16, 16)] = 1.0 / deg
    return carry

  lax.fori_loop(0, DEGW // 16, chunk, 0)

  pltpu.sync_copy(d_v, d_hbm.at[pl.ds(base, DEGW)])
  pltpu.sync_copy(d2_v, d2_hbm.at[pl.ds(base, DEGW)])

  nrows = jnp.minimum(DEGW, jnp.maximum(N - base, 0))

  def row_chunk(i, carry):
    r0 = base + i * 16
    pltpu.async_copy(y_hbm.at[pl.ds(r0, 16)], row_v, sem).wait()

    def scale_row(j, c2):
      s = d_v[i * 16 + j]
      for cc in range(D // 16):
        row_v[j, pl.ds(cc * 16, 16)] = row_v[j, pl.ds(cc * 16, 16)] * s
      return c2

    lax.fori_loop(0, 16, scale_row, 0)
    pltpu.sync_copy(row_v, g0_hbm.at[pl.ds(r0, 16)])
    return carry

  lax.fori_loop(0, nrows // 16, row_chunk, 0)


def _k_prep(degp, y):
  f = pl.kernel(
      _prep_body,
      out_type=(
          jax.ShapeDtypeStruct((NP,), jnp.float32),
          jax.ShapeDtypeStruct((NP,), jnp.float32),
          jax.ShapeDtypeStruct((N, D), jnp.float32),
      ),
      mesh=_mesh(),
      scratch_types=[
          pltpu.VMEM((DEGW,), jnp.float32),
          pltpu.VMEM((DEGW,), jnp.float32),
          pltpu.VMEM((DEGW,), jnp.float32),
          pltpu.VMEM((DEGW,), jnp.float32),
          pltpu.VMEM((16, D), jnp.float32),
          pltpu.SemaphoreType.DMA,
      ],
  )
  return f(degp, y)


def _hop_body(g_hbm, src_hbm, dst_hbm, p_hbm,
              src_v, dst_v, rows_v, sem):
  sid = lax.axis_index("s")
  cid = lax.axis_index("c")
  wid = cid * NS + sid

  def scoped(acc_sh):
    _zero_fill(rows_v.at[0], CHUNK * D)
    for z in range(ROWS_PER_TILE // ZROWS):
      pltpu.sync_copy(
          rows_v.at[0],
          acc_sh.at[pl.ds(sid * ROWS_PER_TILE + z * ZROWS, ZROWS)],
      )

    pltpu.sync_copy(src_hbm.at[wid], src_v)
    pltpu.sync_copy(dst_hbm.at[wid], dst_v)
    plsc.subcore_barrier()

    pltpu.async_copy(g_hbm.at[src_v.at[0]], rows_v.at[0], sem)

    def step(j, carry):
      buf = j % 2

      @pl.when(j + 1 < NCHUNK)
      def _issue_next():
        pltpu.async_copy(g_hbm.at[src_v.at[j + 1]], rows_v.at[1 - buf], sem)

      pltpu.make_async_copy(g_hbm.at[src_v.at[j]], rows_v.at[buf], sem).wait()
      pltpu.sync_copy(rows_v.at[buf], acc_sh.at[dst_v.at[j]], add=True)
      return carry

    lax.fori_loop(0, NCHUNK, step, 0)
    plsc.subcore_barrier()

    pltpu.sync_copy(
        acc_sh.at[pl.ds(sid * ROWS_PER_TILE, ROWS_PER_TILE)],
        p_hbm.at[cid, pl.ds(sid * ROWS_PER_TILE, ROWS_PER_TILE)],
    )

  pl.run_scoped(scoped, pltpu.VMEM_SHARED((N, D), jnp.float32))


def _k_hop(g, src3, dst3):
  f = pl.kernel(
      _hop_body,
      out_type=jax.ShapeDtypeStruct((NC, N, D), jnp.float32),
      mesh=_mesh(),
      scratch_types=[
          pltpu.VMEM((NCHUNK, CHUNK), jnp.int32),
          pltpu.VMEM((NCHUNK, CHUNK), jnp.int32),
          pltpu.VMEM((2, CHUNK, D), jnp.float32),
          pltpu.SemaphoreType.DMA,
      ],
  )
  return f(g, src3, dst3)


NROWCHUNKS = N // 16


def _merge_body(final, p_hbm, g_hbm, s_hbm, b_hbm, out_hbm,
                p0_v, p1_v, g_v, s_v, b_v, sem):
  sid = lax.axis_index("s")
  cid = lax.axis_index("c")
  wid = cid * NS + sid

  if final:
    pltpu.sync_copy(b_hbm, b_v)

  nt = (NROWCHUNKS - wid + NW - 1) // NW

  def chunk(i, carry):
    t = wid + i * NW
    r0 = t * 16
    pltpu.async_copy(p_hbm.at[0, pl.ds(r0, 16)], p0_v, sem).wait()
    pltpu.async_copy(p_hbm.at[1, pl.ds(r0, 16)], p1_v, sem).wait()
    pltpu.async_copy(g_hbm.at[pl.ds(r0, 16)], g_v, sem).wait()
    pltpu.async_copy(s_hbm.at[pl.ds(r0, 16)], s_v, sem).wait()

    def row(j, c2):
      s = s_v[j]
      for cc in range(D // 16):
        v = (p0_v[j, pl.ds(cc * 16, 16)] + p1_v[j, pl.ds(cc * 16, 16)]
             + g_v[j, pl.ds(cc * 16, 16)]) * s
        if final:
          v = v + b_v[pl.ds(cc * 16, 16)]
        g_v[j, pl.ds(cc * 16, 16)] = v
      return c2

    lax.fori_loop(0, 16, row, 0)
    pltpu.sync_copy(g_v, out_hbm.at[pl.ds(r0, 16)])
    return carry

  lax.fori_loop(0, nt, chunk, 0)


def _k_merge(final, p, g, scale, b):
  f = pl.kernel(
      functools.partial(_merge_body, final),
      out_type=jax.ShapeDtypeStruct((N, D), jnp.float32),
      mesh=_mesh(),
      scratch_types=[
          pltpu.VMEM((16, D), jnp.float32),
          pltpu.VMEM((16, D), jnp.float32),
          pltpu.VMEM((16, D), jnp.float32),
          pltpu.VMEM((16,), jnp.float32),
          pltpu.VMEM((D,), jnp.float32),
          pltpu.SemaphoreType.DMA,
      ],
  )
  return f(p, g, scale, b)


def _mm_body(x_ref, w_ref, o_ref):
  o_ref[...] = lax.dot_general(
      x_ref[...], w_ref[...], (((1,), (1,)), ((), ())),
      preferred_element_type=jnp.float32,
  )


def _k_matmul(x, W):
  return pl.pallas_call(
      _mm_body,
      out_shape=jax.ShapeDtypeStruct((N, D), jnp.float32),
  )(x, W)


def kernel(x, edge_index, W, b):
  src = edge_index[0].astype(jnp.int32).reshape(NW, NCHUNK, CHUNK)
  dst = edge_index[1].astype(jnp.int32).reshape(NW, NCHUNK, CHUNK)
  dst_flat = edge_index[1].astype(jnp.int32)

  y = _k_matmul(x, W)
  degp = _k_deg(dst_flat)
  d, d2, g = _k_prep(degp, y)

  for k in range(K_HOPS):
    p = _k_hop(g, src, dst)
    final = k == K_HOPS - 1
    g = _k_merge(final, p, g, d if final else d2, b)

  return g


# speedup vs baseline: 30.5068x; 30.5068x over previous
import functools

import jax
import jax.numpy as jnp
from jax import lax
from jax.experimental import pallas as pl
from jax.experimental.pallas import tpu as pltpu
from jax.experimental.pallas import tpu_sc as plsc

N = 10000
D = 128
E = 320000
K_HOPS = 3

NC = 2
NS = 16
NW = NC * NS

EPW = E // NW
CHUNK = 125
NCHUNK = EPW // CHUNK
BLK = 16
NBLK = NCHUNK // BLK

NP = 10240
DEGW = NP // NW

ACC_ROWS = NP // NS
ZROWS = 80


def _mesh():
  return plsc.VectorSubcoreMesh(
      core_axis_name="c", subcore_axis_name="s", num_cores=NC, num_subcores=NS
  )


def _zero_fill(vref, nwords):
  z = jnp.zeros((16,), jnp.float32)

  def body(i, carry):
    vref[pl.ds(i * 16, 16)] = z
    return carry

  lax.fori_loop(0, nwords // 16, body, 0)


def _zero_fill_2d(vref, nrows):
  z = jnp.zeros((16,), jnp.float32)

  def body(r, carry):
    for c in range(D // 16):
      vref[r, pl.ds(c * 16, 16)] = z
    return carry

  lax.fori_loop(0, nrows, body, 0)


def _deg_body(dst_hbm, degp_hbm, idx_v, ones_v, zbuf_v, deg_sh):
  sid = lax.axis_index("s")
  cid = lax.axis_index("c")
  wid = cid * NS + sid

  zslice = NP // NS
  _zero_fill(zbuf_v, zslice)
  pltpu.sync_copy(zbuf_v, deg_sh.at[pl.ds(sid * zslice, zslice)])

  one = jnp.ones((16,), jnp.float32)
  for i in range(8):
    ones_v[pl.ds(i * 16, 16)] = one

  pltpu.sync_copy(dst_hbm.at[wid], idx_v)
  plsc.subcore_barrier()

  def step(j, carry):
    pltpu.sync_copy(ones_v.at[pl.ds(0, CHUNK)], deg_sh.at[idx_v.at[j]],
                    add=True)
    return carry

  lax.fori_loop(0, NCHUNK, step, 0)
  plsc.subcore_barrier()

  pltpu.sync_copy(
      deg_sh.at[pl.ds(sid * zslice, zslice)],
      degp_hbm.at[pl.ds(cid * NP + sid * zslice, zslice)],
  )


def _k_deg(dst3):
  f = pl.kernel(
      _deg_body,
      out_type=jax.ShapeDtypeStruct((NC * NP,), jnp.float32),
      mesh=_mesh(),
      scratch_types=[
          pltpu.VMEM((NCHUNK, CHUNK), jnp.int32),
          pltpu.VMEM((128,), jnp.float32),
          pltpu.VMEM((NP // NS,), jnp.float32),
          pltpu.VMEM_SHARED((NP,), jnp.float32),
      ],
  )
  return f(dst3)


def _hop_body(g_hbm, src_hbm, dst_hbm, p_hbm,
              src_v, dst_v, rows_v, sem, acc_sh):
  sid = lax.axis_index("s")
  cid = lax.axis_index("c")
  wid = cid * NS + sid

  _zero_fill_2d(rows_v.at[0], ZROWS)
  for z in range(ACC_ROWS // ZROWS):
    pltpu.sync_copy(
        rows_v.at[0, pl.ds(0, ZROWS)],
        acc_sh.at[pl.ds(sid * ACC_ROWS + z * ZROWS, ZROWS)],
    )

  plsc.subcore_barrier()

  def block(blk, carry):
    pltpu.sync_copy(src_hbm.at[wid, pl.ds(blk * BLK, BLK)], src_v)
    pltpu.sync_copy(dst_hbm.at[wid, pl.ds(blk * BLK, BLK)], dst_v)
    pltpu.async_copy(g_hbm.at[src_v.at[0]], rows_v.at[0], sem)

    def step(t, c2):
      buf = t % 2

      @pl.when(t + 1 < BLK)
      def _issue_next():
        pltpu.async_copy(g_hbm.at[src_v.at[t + 1]], rows_v.at[1 - buf], sem)

      pltpu.make_async_copy(g_hbm.at[src_v.at[t]], rows_v.at[buf], sem).wait()
      pltpu.sync_copy(rows_v.at[buf], acc_sh.at[dst_v.at[t]], add=True)
      return c2

    lax.fori_loop(0, BLK, step, 0)
    return carry

  lax.fori_loop(0, NBLK, block, 0)
  plsc.subcore_barrier()

  pltpu.sync_copy(
      acc_sh.at[pl.ds(sid * ACC_ROWS, ACC_ROWS)],
      p_hbm.at[cid, pl.ds(sid * ACC_ROWS, ACC_ROWS)],
  )


def _k_hop(g, src3, dst3):
  f = pl.kernel(
      _hop_body,
      out_type=jax.ShapeDtypeStruct((NC, NP, D), jnp.float32),
      mesh=_mesh(),
      scratch_types=[
          pltpu.VMEM((BLK, CHUNK), jnp.int32),
          pltpu.VMEM((BLK, CHUNK), jnp.int32),
          pltpu.VMEM((2, CHUNK, D), jnp.float32),
          pltpu.SemaphoreType.DMA,
          pltpu.VMEM_SHARED((NP, D), jnp.float32),
      ],
  )
  return f(g, src3, dst3)


def _merge_body(final, p_ref, g_ref, s_ref, b_ref, out_ref):
  v = (p_ref[0, :N, :] + p_ref[1, :N, :] + g_ref[...]) * s_ref[:N][:, None]
  if final:
    v = v + b_ref[...][None, :]
  out_ref[...] = v


def _k_merge(final, p, g, scale, b):
  return pl.pallas_call(
      functools.partial(_merge_body, final),
      out_shape=jax.ShapeDtypeStruct((N, D), jnp.float32),
  )(p, g, scale, b)


def _pre_body(x_ref, w_ref, degp_ref, g0_ref, d_ref, d2_ref):
  deg = degp_ref[0, :] + degp_ref[1, :] + 1.0
  d = lax.rsqrt(deg)
  d_ref[...] = d
  d2_ref[...] = 1.0 / deg
  y = lax.dot_general(
      x_ref[...], w_ref[...], (((1,), (1,)), ((), ())),
      preferred_element_type=jnp.float32,
  )
  g0_ref[...] = y * d[:N, None]


def _k_pre(x, W, degp):
  return pl.pallas_call(
      _pre_body,
      out_shape=(
          jax.ShapeDtypeStruct((N, D), jnp.float32),
          jax.ShapeDtypeStruct((NP,), jnp.float32),
          jax.ShapeDtypeStruct((NP,), jnp.float32),
      ),
  )(x, W, degp.reshape(NC, NP))


def kernel(x, edge_index, W, b):
  src = edge_index[0].astype(jnp.int32).reshape(NW, NCHUNK, CHUNK)
  dst = edge_index[1].astype(jnp.int32).reshape(NW, NCHUNK, CHUNK)

  degp = _k_deg(dst)
  g, d, d2 = _k_pre(x, W, degp)

  for k in range(K_HOPS):
    p = _k_hop(g, src, dst)
    final = k == K_HOPS - 1
    g = _k_merge(final, p, g, d if final else d2, b)

  return g
